# baseline (speedup 1.0000x reference)
import functools

import jax
import jax.numpy as jnp
from jax import lax
from jax.experimental import pallas as pl
from jax.experimental.pallas import tpu as pltpu

_DN = (((1,), (1,)), ((), ()))


def _dist_min_kernel(y_ref, x_ref, out_ref, tot_ref, acc_ref, bt_ref):
    j = pl.program_id(0)
    nk = pl.num_programs(0)

    @pl.when(j == 0)
    def _init():
        acc_ref[...] = jnp.full(acc_ref.shape, jnp.inf, acc_ref.dtype)
        x = x_ref[...]
        bt_ref[...] = jnp.concatenate(
            [(-2.0 * x).astype(jnp.bfloat16),
             jnp.ones(x.shape, jnp.bfloat16)], axis=1)

    yy = y_ref[...].astype(jnp.bfloat16)
    bt = bt_ref[...]
    dim = yy.shape[1] // 2
    m8 = acc_ref[...]
    for half in (yy[:, :dim], yy[:, dim:]):
        a = jnp.concatenate([half, half * half], axis=1)
        d = lax.dot_general(a, bt, _DN,
                            preferred_element_type=jnp.float32)
        m8 = jnp.minimum(m8, jnp.min(d.reshape(-1, 8, d.shape[1]), axis=0))
    acc_ref[...] = m8

    @pl.when(j == nk - 1)
    def _finish():
        x = x_ref[...]
        x2 = lax.dot_general(jnp.ones((1, x.shape[1]), jnp.float32), x * x,
                             _DN, preferred_element_type=jnp.float32)
        r = jnp.min(acc_ref[...], axis=0, keepdims=True) + x2
        out_ref[...] = r
        tot_ref[...] = jnp.sum(r).reshape(1, 1)


@functools.partial(jax.jit, static_argnames=())
def kernel(x, y):
    q, dim = x.shape
    k = y.shape[0]
    y2w = y.reshape(k // 2, 2 * dim)
    kb2 = 2000
    nk = (k // 2) // kb2
    assert nk * kb2 == k // 2

    closest_row, tot = pl.pallas_call(
        _dist_min_kernel,
        grid=(nk,),
        in_specs=[
            pl.BlockSpec((kb2, 2 * dim), lambda j: (j, 0)),
            pl.BlockSpec((q, dim), lambda j: (0, 0)),
        ],
        out_specs=[
            pl.BlockSpec((1, q), lambda j: (0, 0)),
            pl.BlockSpec((1, 1), lambda j: (0, 0)),
        ],
        out_shape=[
            jax.ShapeDtypeStruct((1, q), jnp.float32),
            jax.ShapeDtypeStruct((1, 1), jnp.float32),
        ],
        scratch_shapes=[
            pltpu.VMEM((8, q), jnp.float32),
            pltpu.VMEM((q, 2 * dim), jnp.bfloat16),
        ],
        compiler_params=pltpu.CompilerParams(
            dimension_semantics=("arbitrary",),
        ),
    )(y2w, x)

    return (tot.reshape(()), closest_row.reshape(q))


# speedup vs baseline: 1.4637x; 1.4637x over previous
import functools

import jax
import jax.numpy as jnp
from jax import lax
from jax.experimental import pallas as pl
from jax.experimental.pallas import tpu as pltpu

_DN = (((1,), (1,)), ((), ()))


def _dist_min_kernel(y_ref, x_ref, out_ref, tot_ref, acc_ref, bt_ref):
    j = pl.program_id(0)
    nk = pl.num_programs(0)

    @pl.when(j == 0)
    def _init():
        acc_ref[...] = jnp.full(acc_ref.shape, jnp.inf, acc_ref.dtype)
        x = x_ref[...]
        bt_ref[...] = jnp.concatenate(
            [(-2.0 * x).astype(jnp.bfloat16),
             jnp.ones(x.shape, jnp.bfloat16)], axis=1)

    y_blk = y_ref[...].astype(jnp.bfloat16)
    a = jnp.concatenate([y_blk, y_blk * y_blk], axis=1)
    d = lax.dot_general(a, bt_ref[...], _DN,
                        preferred_element_type=jnp.float32)
    m8 = jnp.min(d.reshape(-1, 8, d.shape[1]), axis=0)
    acc_ref[...] = jnp.minimum(acc_ref[...], m8)

    @pl.when(j == nk - 1)
    def _finish():
        x = x_ref[...]
        x2 = lax.dot_general(jnp.ones((1, x.shape[1]), jnp.float32), x * x,
                             _DN, preferred_element_type=jnp.float32)
        r = jnp.min(acc_ref[...], axis=0, keepdims=True) + x2
        out_ref[...] = r
        tot_ref[...] = jnp.sum(r).reshape(1, 1)


@functools.partial(jax.jit, static_argnames=())
def kernel(x, y):
    q, dim = x.shape
    k = y.shape[0]
    kb = 10000
    nk = k // kb
    assert nk * kb == k

    closest_row, tot = pl.pallas_call(
        _dist_min_kernel,
        grid=(nk,),
        in_specs=[
            pl.BlockSpec((kb, dim), lambda j: (j, 0)),
            pl.BlockSpec((q, dim), lambda j: (0, 0)),
        ],
        out_specs=[
            pl.BlockSpec((1, q), lambda j: (0, 0)),
            pl.BlockSpec((1, 1), lambda j: (0, 0)),
        ],
        out_shape=[
            jax.ShapeDtypeStruct((1, q), jnp.float32),
            jax.ShapeDtypeStruct((1, 1), jnp.float32),
        ],
        scratch_shapes=[
            pltpu.VMEM((8, q), jnp.float32),
            pltpu.VMEM((q, 2 * dim), jnp.bfloat16),
        ],
        compiler_params=pltpu.CompilerParams(
            dimension_semantics=("arbitrary",),
        ),
    )(y, x)

    return (tot.reshape(()), closest_row.reshape(q))
